# baseline (speedup 1.0000x reference)
import functools
import math

import jax
import jax.numpy as jnp
from jax.experimental import pallas as pl
from jax.experimental.pallas import tpu as pltpu

_B, _S, _DIM = 1, 2048, 2048
_H, _DH = 16, 128
_RQ = 512
_G, _RO = 4, 128
_EPS = 1e-6
_BQ = 256
_LOG2E = 1.4426950408889634


def _dot(a, b, dims):
    return jax.lax.dot_general(a, b, (dims, ((), ())),
                               preferred_element_type=jnp.float32)


def _body(x_ref, wqd_ref, qln_ref, wqu_ref, wkv_ref, kvln_ref, sink_ref,
          wod_ref, wou_ref, o_ref, kv_scr):
    i = pl.program_id(0)
    xb = x_ref[...]

    kvh = _dot(xb, wkv_ref[...], ((1,), (1,)))
    var = jnp.mean(kvh * kvh, axis=-1, keepdims=True)
    kvn = kvh * jax.lax.rsqrt(var + _EPS) * kvln_ref[...]
    kv_scr[pl.ds(i * _BQ, _BQ), :] = kvn.astype(jnp.bfloat16)

    qh = _dot(xb, wqd_ref[...], ((1,), (1,)))
    qvar = jnp.mean(qh * qh, axis=-1, keepdims=True)
    qn = (qh * jax.lax.rsqrt(qvar + _EPS) * qln_ref[...]).astype(jnp.bfloat16)
    qb = _dot(qn, wqu_ref[...], ((1,), (1,)))
    qbs = (qb * (_LOG2E / math.sqrt(_DH))).astype(jnp.bfloat16)

    kv_d = kv_scr[pl.ds(i * _BQ, _BQ), :]
    r_loc = jax.lax.broadcasted_iota(jnp.int32, (_BQ, _BQ), 0)
    c_loc = jax.lax.broadcasted_iota(jnp.int32, (_BQ, _BQ), 1)
    tri = c_loc <= r_loc
    esink = jax.lax.exp2(sink_ref[...] * _LOG2E)

    parts = []
    for h in range(_H):
        q_h = qbs[:, h * _DH:(h + 1) * _DH]
        e_d = jnp.where(tri, jax.lax.exp2(_dot(q_h, kv_d, ((1,), (1,)))), 0.0)
        denom0 = jnp.sum(e_d, axis=-1, keepdims=True) + esink[0, h]
        acc0 = _dot(e_d.astype(jnp.bfloat16), kv_d, ((1,), (0,)))

        def chunk(j, carry, q_h=q_h):
            denom, acc = carry
            kv_j = kv_scr[pl.ds(j * _BQ, _BQ), :]
            e = jax.lax.exp2(_dot(q_h, kv_j, ((1,), (1,))))
            denom = denom + jnp.sum(e, axis=-1, keepdims=True)
            acc = acc + _dot(e.astype(jnp.bfloat16), kv_j, ((1,), (0,)))
            return denom, acc

        denom, acc = jax.lax.fori_loop(0, i, chunk, (denom0, acc0))
        parts.append(acc / denom)
    att = jnp.concatenate(parts, axis=1)

    z_parts = []
    for g in range(_G):
        og = att[:, g * (_H // _G) * _DH:(g + 1) * (_H // _G) * _DH]
        wdg = wod_ref[g * _RO:(g + 1) * _RO, :]
        z_parts.append(_dot(og.astype(jnp.bfloat16), wdg, ((1,), (1,))))
    z = jnp.concatenate(z_parts, axis=1).astype(jnp.bfloat16)
    o_ref[...] = _dot(z, wou_ref[...], ((1,), (1,)))


@functools.partial(jax.jit, static_argnames=())
def kernel(x, wq_down, q_ln, wq_up, wkv, kv_ln, attn_sink, wo_down, wo_up):
    xs = x.reshape(_S, _DIM).astype(jnp.bfloat16)
    full = lambda shape: pl.BlockSpec(shape, lambda i: (0, 0))
    out = pl.pallas_call(
        _body,
        grid=(_S // _BQ,),
        in_specs=[
            pl.BlockSpec((_BQ, _DIM), lambda i: (i, 0)),
            full((_RQ, _DIM)),
            full((1, _RQ)),
            full((_H * _DH, _RQ)),
            full((_DH, _DIM)),
            full((1, _DH)),
            full((1, _H)),
            full((_G * _RO, (_H * _DH) // _G)),
            full((_DIM, _G * _RO)),
        ],
        out_specs=pl.BlockSpec((_BQ, _DIM), lambda i: (i, 0)),
        out_shape=jax.ShapeDtypeStruct((_S, _DIM), jnp.float32),
        scratch_shapes=[pltpu.VMEM((_S, _DH), jnp.bfloat16)],
        compiler_params=pltpu.CompilerParams(
            dimension_semantics=("arbitrary",)),
    )(
        xs,
        wq_down.astype(jnp.bfloat16),
        q_ln.reshape(1, _RQ),
        wq_up.astype(jnp.bfloat16),
        wkv.astype(jnp.bfloat16),
        kv_ln.reshape(1, _DH),
        attn_sink.reshape(1, _H),
        wo_down.astype(jnp.bfloat16),
        wo_up.astype(jnp.bfloat16),
    )
    return out.reshape(_B, _S, _DIM)


# speedup vs baseline: 2.0502x; 2.0502x over previous
import functools
import math

import jax
import jax.numpy as jnp
from jax.experimental import pallas as pl
from jax.experimental.pallas import tpu as pltpu

_B, _S, _DIM = 1, 2048, 2048
_H, _DH = 16, 128
_RQ = 512
_G, _RO = 4, 128
_EPS = 1e-6
_BQ = 256
_LOG2E = 1.4426950408889634


def _dot(a, b, dims):
    return jax.lax.dot_general(a, b, (dims, ((), ())),
                               preferred_element_type=jnp.float32)


def _body(x_ref, wqd_ref, qln_ref, wqu_ref, wkv_ref, kvln_ref, sink_ref,
          wod_ref, wou_ref, o_ref, kv_scr):
    i = pl.program_id(0)

    @pl.when(i == 0)
    def _init():
        kv_scr[...] = jnp.zeros((_S, _DH), jnp.bfloat16)

    xb = x_ref[...]

    kvh = _dot(xb, wkv_ref[...], ((1,), (1,)))
    var = jnp.mean(kvh * kvh, axis=-1, keepdims=True)
    kvn = kvh * jax.lax.rsqrt(var + _EPS) * kvln_ref[...]
    kv_scr[pl.ds(i * _BQ, _BQ), :] = kvn.astype(jnp.bfloat16)

    qh = _dot(xb, wqd_ref[...], ((1,), (1,)))
    qvar = jnp.mean(qh * qh, axis=-1, keepdims=True)
    qn = (qh * jax.lax.rsqrt(qvar + _EPS) * qln_ref[...]).astype(jnp.bfloat16)
    qb = _dot(qn, wqu_ref[...], ((1,), (1,)))
    qbs = (qb * (_LOG2E / math.sqrt(_DH))).astype(jnp.bfloat16)

    kv_all = kv_scr[...]
    rows = i * _BQ + jax.lax.broadcasted_iota(jnp.int32, (_BQ, _S), 0)
    cols = jax.lax.broadcasted_iota(jnp.int32, (_BQ, _S), 1)
    mask = cols <= rows
    esink = jax.lax.exp2(sink_ref[...] * _LOG2E)

    parts = []
    for h in range(_H):
        q_h = qbs[:, h * _DH:(h + 1) * _DH]
        logits = _dot(q_h, kv_all, ((1,), (1,)))
        e = jnp.where(mask, jax.lax.exp2(logits), 0.0)
        denom = jnp.sum(e, axis=-1, keepdims=True) + esink[0, h]
        acc = _dot(e.astype(jnp.bfloat16), kv_all, ((1,), (0,)))
        parts.append(acc / denom)
    att = jnp.concatenate(parts, axis=1)

    z_parts = []
    for g in range(_G):
        og = att[:, g * (_H // _G) * _DH:(g + 1) * (_H // _G) * _DH]
        wdg = wod_ref[g * _RO:(g + 1) * _RO, :]
        z_parts.append(_dot(og.astype(jnp.bfloat16), wdg, ((1,), (1,))))
    z = jnp.concatenate(z_parts, axis=1).astype(jnp.bfloat16)
    o_ref[...] = _dot(z, wou_ref[...], ((1,), (1,)))


@functools.partial(jax.jit, static_argnames=())
def kernel(x, wq_down, q_ln, wq_up, wkv, kv_ln, attn_sink, wo_down, wo_up):
    xs = x.reshape(_S, _DIM).astype(jnp.bfloat16)
    full = lambda shape: pl.BlockSpec(shape, lambda i: (0, 0))
    out = pl.pallas_call(
        _body,
        grid=(_S // _BQ,),
        in_specs=[
            pl.BlockSpec((_BQ, _DIM), lambda i: (i, 0)),
            full((_RQ, _DIM)),
            full((1, _RQ)),
            full((_H * _DH, _RQ)),
            full((_DH, _DIM)),
            full((1, _DH)),
            full((1, _H)),
            full((_G * _RO, (_H * _DH) // _G)),
            full((_DIM, _G * _RO)),
        ],
        out_specs=pl.BlockSpec((_BQ, _DIM), lambda i: (i, 0)),
        out_shape=jax.ShapeDtypeStruct((_S, _DIM), jnp.float32),
        scratch_shapes=[pltpu.VMEM((_S, _DH), jnp.bfloat16)],
        compiler_params=pltpu.CompilerParams(
            dimension_semantics=("arbitrary",)),
    )(
        xs,
        wq_down.astype(jnp.bfloat16),
        q_ln.reshape(1, _RQ),
        wq_up.astype(jnp.bfloat16),
        wkv.astype(jnp.bfloat16),
        kv_ln.reshape(1, _DH),
        attn_sink.reshape(1, _H),
        wo_down.astype(jnp.bfloat16),
        wo_up.astype(jnp.bfloat16),
    )
    return out.reshape(_B, _S, _DIM)
